# Initial kernel scaffold; baseline (speedup 1.0000x reference)
import functools

import jax
import jax.numpy as jnp
from jax import lax
from jax.experimental import pallas as pl
from jax.experimental.pallas import tpu as pltpu
from jax.experimental.pallas import tpu_sc as plsc

DEG = 64
HIDDEN = 128
LANES = 16
N_CHUNKS = HIDDEN // LANES


def _sc_body(v_hbm, idx_hbm, times_hbm, thr_hbm, out_hbm,
             idx_v, times_v, thr_v, maskf_v, rows_v, out_v, sem):
    worker = jnp.logical_and(lax.axis_index("c") == 0, lax.axis_index("s") == 0)

    @pl.when(worker)
    def _():
        pltpu.sync_copy(idx_hbm, idx_v)
        gather = pltpu.async_copy(v_hbm.at[idx_v], rows_v, sem)
        pltpu.sync_copy(times_hbm, times_v)
        pltpu.sync_copy(thr_hbm, thr_v)
        thr = thr_v[...]
        for c in range(DEG // LANES):
            tv = times_v[pl.ds(c * LANES, LANES)]
            maskf_v[pl.ds(c * LANES, LANES)] = jnp.where(
                tv <= thr, jnp.float32(1.0), jnp.float32(0.0))
        gather.wait()
        acc = [jnp.zeros((LANES,), jnp.float32) for _ in range(N_CHUNKS)]
        for i in range(DEG):
            mi = plsc.load_gather(maskf_v, [jnp.full((LANES,), i, jnp.int32)])
            for j in range(N_CHUNKS):
                acc[j] = acc[j] + mi * rows_v[i, pl.ds(j * LANES, LANES)]
        for j in range(N_CHUNKS):
            out_v[pl.ds(j * LANES, LANES)] = acc[j]
        pltpu.sync_copy(out_v, out_hbm)


def _sc_gather_sum(v, idx, times_flat, thr16):
    mesh = plsc.VectorSubcoreMesh(core_axis_name="c", subcore_axis_name="s")
    return pl.kernel(
        _sc_body,
        out_type=jax.ShapeDtypeStruct((HIDDEN,), jnp.float32),
        mesh=mesh,
        scratch_types=[
            pltpu.VMEM((DEG,), jnp.int32),
            pltpu.VMEM((DEG,), jnp.float32),
            pltpu.VMEM((LANES,), jnp.float32),
            pltpu.VMEM((DEG,), jnp.float32),
            pltpu.VMEM((DEG, HIDDEN), jnp.float32),
            pltpu.VMEM((HIDDEN,), jnp.float32),
            pltpu.SemaphoreType.DMA,
        ],
    )(v, idx, times_flat, thr16)


def _tc_body(times_ref, thr_ref, w0_ref, b0_ref, W_ref, B_ref, Wv_ref, bv_ref,
             out_ref):
    times = times_ref[...]
    maskf = (times <= thr_ref[...]).astype(jnp.float32)
    lin = times * w0_ref[...] + b0_ref[...]
    per = jnp.sin(times * W_ref[...] + B_ref[...])
    z = jnp.concatenate([lin, per], axis=1)
    zm = z * maskf
    t_v_sum = jnp.sum(
        jnp.dot(zm, Wv_ref[...], preferred_element_type=jnp.float32),
        axis=0, keepdims=True)
    cnt = jnp.sum(maskf)
    out_ref[...] = t_v_sum + cnt * bv_ref[...]


def _tc_time_term(times, thr, w0, b0, W, B, Wv, bv):
    return pl.pallas_call(
        _tc_body,
        out_shape=jax.ShapeDtypeStruct((1, HIDDEN), jnp.float32),
    )(times, thr, w0, b0, W, B, Wv, bv)


def kernel(nid, k, q, v, t, neighbors, times,
           t2v_w0, t2v_b0, t2v_W, t2v_B, Wq, bq, Wk, bk, Wv, bv):
    del nid, k, q, Wq, bq, Wk, bk
    sel = neighbors.reshape(DEG).astype(jnp.int32)
    times_flat = times.reshape(DEG).astype(jnp.float32)
    thr16 = jnp.broadcast_to(t.reshape(1), (LANES,)).astype(jnp.float32)
    row_sum = _sc_gather_sum(v, sel, times_flat, thr16)
    term = _tc_time_term(
        times.reshape(DEG, 1), t.reshape(1, 1), t2v_w0.reshape(1, 1),
        t2v_b0.reshape(1, 1), t2v_W.reshape(1, 15), t2v_B.reshape(1, 15),
        Wv, bv.reshape(1, HIDDEN))
    return row_sum.reshape(1, HIDDEN) + term


# speedup vs baseline: 1.4444x; 1.4444x over previous
import jax
import jax.numpy as jnp
from jax import lax
from jax.experimental import pallas as pl
from jax.experimental.pallas import tpu as pltpu
from jax.experimental.pallas import tpu_sc as plsc

DEG = 64
HIDDEN = 128
LANES = 16
N_CHUNKS = HIDDEN // LANES
GATHER_ROWS = DEG + LANES


def _sc_body(v_hbm, idx_hbm, cnt_hbm, out_hbm, idx_v, cnt_v, rows_v, out_v, sem):
    worker = jnp.logical_and(lax.axis_index("c") == 0, lax.axis_index("s") == 0)

    @pl.when(worker)
    def _():
        pltpu.sync_copy(idx_hbm, idx_v)
        pltpu.sync_copy(cnt_hbm, cnt_v)
        pltpu.async_copy(v_hbm.at[idx_v], rows_v, sem).wait()
        cnt0 = cnt_v[...]
        acc = [jnp.zeros((LANES,), jnp.float32) for _ in range(N_CHUNKS)]
        for i in range(DEG):
            for j in range(N_CHUNKS):
                acc[j] = acc[j] + rows_v[i, pl.ds(j * LANES, LANES)]
        for j in range(N_CHUNKS):
            sl = pl.ds(j * LANES, LANES)
            out_v[sl] = acc[j] - cnt0 * rows_v[DEG, sl]
        pltpu.sync_copy(out_v, out_hbm)


def _sc_gather_sum(v, idx80, cnt16):
    mesh = plsc.VectorSubcoreMesh(core_axis_name="c", subcore_axis_name="s")
    return pl.kernel(
        _sc_body,
        out_type=jax.ShapeDtypeStruct((HIDDEN,), jnp.float32),
        mesh=mesh,
        scratch_types=[
            pltpu.VMEM((GATHER_ROWS,), jnp.int32),
            pltpu.VMEM((LANES,), jnp.float32),
            pltpu.VMEM((GATHER_ROWS, HIDDEN), jnp.float32),
            pltpu.VMEM((HIDDEN,), jnp.float32),
            pltpu.SemaphoreType.DMA,
        ],
    )(v, idx80, cnt16)


def _tc_body(times_ref, thr_ref, sel_ref, w0_ref, b0_ref, W_ref, B_ref,
             Wv_ref, bv_ref, term_ref, idx_ref, cnt_ref):
    times = times_ref[...]
    mask = times <= thr_ref[...]
    maskf = mask.astype(jnp.float32)
    selp = jnp.where(mask, sel_ref[...], jnp.zeros((DEG, 1), jnp.int32))
    idx_ref[...] = jnp.concatenate(
        [selp, jnp.zeros((LANES, 1), jnp.int32)], axis=0)
    cnt_ref[...] = jnp.broadcast_to(jnp.sum(jnp.float32(1.0) - maskf),
                                    (1, LANES))
    lin = times * w0_ref[...] + b0_ref[...]
    per = jnp.sin(times * W_ref[...] + B_ref[...])
    z = jnp.concatenate([lin, per], axis=1)
    zm = z * maskf
    t_v_sum = jnp.sum(
        jnp.dot(zm, Wv_ref[...], preferred_element_type=jnp.float32),
        axis=0, keepdims=True)
    cnt_in = jnp.sum(maskf)
    term_ref[...] = t_v_sum + cnt_in * bv_ref[...]


def _tc_time_term(times, thr, sel, w0, b0, W, B, Wv, bv):
    return pl.pallas_call(
        _tc_body,
        out_shape=(
            jax.ShapeDtypeStruct((1, HIDDEN), jnp.float32),
            jax.ShapeDtypeStruct((GATHER_ROWS, 1), jnp.int32),
            jax.ShapeDtypeStruct((1, LANES), jnp.float32),
        ),
    )(times, thr, sel, w0, b0, W, B, Wv, bv)


def kernel(nid, k, q, v, t, neighbors, times,
           t2v_w0, t2v_b0, t2v_W, t2v_B, Wq, bq, Wk, bk, Wv, bv):
    del nid, k, q, Wq, bq, Wk, bk
    term, idx80, cnt16 = _tc_time_term(
        times.reshape(DEG, 1), t.reshape(1, 1),
        neighbors.reshape(DEG, 1).astype(jnp.int32), t2v_w0.reshape(1, 1),
        t2v_b0.reshape(1, 1), t2v_W.reshape(1, 15), t2v_B.reshape(1, 15),
        Wv, bv.reshape(1, HIDDEN))
    row_sum = _sc_gather_sum(v, idx80.reshape(GATHER_ROWS),
                             cnt16.reshape(LANES))
    return row_sum.reshape(1, HIDDEN) + term


# speedup vs baseline: 1.5225x; 1.0541x over previous
import jax
import jax.numpy as jnp
from jax import lax
from jax.experimental import pallas as pl
from jax.experimental.pallas import tpu as pltpu
from jax.experimental.pallas import tpu_sc as plsc

DEG = 64
HIDDEN = 128
LANES = 16
N_CHUNKS = HIDDEN // LANES
GATHER_ROWS = DEG + LANES


def _sc_body(v_hbm, idx_hbm, cnt_hbm, term_hbm, out_hbm,
             idx_v, cnt_v, term_v, rows_v, out_v, sem):
    worker = jnp.logical_and(lax.axis_index("c") == 0, lax.axis_index("s") == 0)

    @pl.when(worker)
    def _():
        pltpu.sync_copy(idx_hbm, idx_v)
        gather = pltpu.async_copy(v_hbm.at[idx_v], rows_v, sem)
        pltpu.sync_copy(cnt_hbm, cnt_v)
        pltpu.sync_copy(term_hbm, term_v)
        gather.wait()
        cnt0 = cnt_v[...]
        acc = [jnp.zeros((LANES,), jnp.float32) for _ in range(N_CHUNKS)]
        for i in range(DEG):
            for j in range(N_CHUNKS):
                acc[j] = acc[j] + rows_v[i, pl.ds(j * LANES, LANES)]
        for j in range(N_CHUNKS):
            sl = pl.ds(j * LANES, LANES)
            out_v[sl] = acc[j] - cnt0 * rows_v[DEG, sl] + term_v[sl]
        pltpu.sync_copy(out_v, out_hbm)


def _sc_gather_sum(v, idx80, cnt16, term128):
    mesh = plsc.VectorSubcoreMesh(core_axis_name="c", subcore_axis_name="s")
    return pl.kernel(
        _sc_body,
        out_type=jax.ShapeDtypeStruct((HIDDEN,), jnp.float32),
        mesh=mesh,
        scratch_types=[
            pltpu.VMEM((GATHER_ROWS,), jnp.int32),
            pltpu.VMEM((LANES,), jnp.float32),
            pltpu.VMEM((HIDDEN,), jnp.float32),
            pltpu.VMEM((GATHER_ROWS, HIDDEN), jnp.float32),
            pltpu.VMEM((HIDDEN,), jnp.float32),
            pltpu.SemaphoreType.DMA,
        ],
    )(v, idx80, cnt16, term128)


def _tc_body(times_ref, thr_ref, sel_ref, w0_ref, b0_ref, W_ref, B_ref,
             Wv_ref, bv_ref, term_ref, idx_ref, cnt_ref):
    times = times_ref[...]
    mask = times <= thr_ref[...]
    maskf = mask.astype(jnp.float32)
    selp = jnp.where(mask, sel_ref[...], jnp.zeros((DEG, 1), jnp.int32))
    idx_ref[...] = jnp.concatenate(
        [selp, jnp.zeros((LANES, 1), jnp.int32)], axis=0)
    cnt_ref[...] = jnp.broadcast_to(jnp.sum(jnp.float32(1.0) - maskf),
                                    (1, LANES))
    lin = times * w0_ref[...] + b0_ref[...]
    per = jnp.sin(times * W_ref[...] + B_ref[...])
    z = jnp.concatenate([lin, per], axis=1)
    zm = z * maskf
    t_v_sum = jnp.sum(
        jnp.dot(zm, Wv_ref[...], preferred_element_type=jnp.float32),
        axis=0, keepdims=True)
    cnt_in = jnp.sum(maskf)
    term_ref[...] = t_v_sum + cnt_in * bv_ref[...]


def _tc_time_term(times, thr, sel, w0, b0, W, B, Wv, bv):
    return pl.pallas_call(
        _tc_body,
        out_shape=(
            jax.ShapeDtypeStruct((1, HIDDEN), jnp.float32),
            jax.ShapeDtypeStruct((GATHER_ROWS, 1), jnp.int32),
            jax.ShapeDtypeStruct((1, LANES), jnp.float32),
        ),
    )(times, thr, sel, w0, b0, W, B, Wv, bv)


def kernel(nid, k, q, v, t, neighbors, times,
           t2v_w0, t2v_b0, t2v_W, t2v_B, Wq, bq, Wk, bk, Wv, bv):
    del nid, k, q, Wq, bq, Wk, bk
    term, idx80, cnt16 = _tc_time_term(
        times.reshape(DEG, 1), t.reshape(1, 1),
        neighbors.reshape(DEG, 1).astype(jnp.int32), t2v_w0.reshape(1, 1),
        t2v_b0.reshape(1, 1), t2v_W.reshape(1, 15), t2v_B.reshape(1, 15),
        Wv, bv.reshape(1, HIDDEN))
    out = _sc_gather_sum(v, idx80.reshape(GATHER_ROWS), cnt16.reshape(LANES),
                         term.reshape(HIDDEN))
    return out.reshape(1, HIDDEN)


# speedup vs baseline: 1.6925x; 1.1116x over previous
import jax
import jax.numpy as jnp
from jax import lax
from jax.experimental import pallas as pl
from jax.experimental.pallas import tpu as pltpu
from jax.experimental.pallas import tpu_sc as plsc

DEG = 64
HIDDEN = 128
LANES = 16
N_CHUNKS = HIDDEN // LANES


def _sc_body(v_hbm, sel_hbm, times_hbm, thr_hbm, out_hbm,
             idx_v, times_v, thr_v, rows_v, out_v, sem):
    worker = jnp.logical_and(lax.axis_index("c") == 0, lax.axis_index("s") == 0)

    @pl.when(worker)
    def _():
        pltpu.sync_copy(sel_hbm, idx_v)
        pltpu.sync_copy(times_hbm, times_v)
        pltpu.sync_copy(thr_hbm, thr_v)
        thr = thr_v[...]
        for c in range(DEG // LANES):
            sl = pl.ds(c * LANES, LANES)
            m = times_v[sl] <= thr
            idx_v[sl] = jnp.where(m, idx_v[sl], jnp.zeros((LANES,), jnp.int32))
        pltpu.async_copy(v_hbm.at[idx_v], rows_v, sem).wait()
        acc = [jnp.zeros((LANES,), jnp.float32) for _ in range(N_CHUNKS)]
        for i in range(DEG):
            for j in range(N_CHUNKS):
                acc[j] = acc[j] + rows_v[i, pl.ds(j * LANES, LANES)]
        for j in range(N_CHUNKS):
            out_v[pl.ds(j * LANES, LANES)] = acc[j]
        pltpu.sync_copy(out_v, out_hbm)


def _sc_gather_sum(v, sel, times_flat, thr16):
    mesh = plsc.VectorSubcoreMesh(core_axis_name="c", subcore_axis_name="s")
    return pl.kernel(
        _sc_body,
        out_type=jax.ShapeDtypeStruct((HIDDEN,), jnp.float32),
        mesh=mesh,
        scratch_types=[
            pltpu.VMEM((DEG,), jnp.int32),
            pltpu.VMEM((DEG,), jnp.float32),
            pltpu.VMEM((LANES,), jnp.float32),
            pltpu.VMEM((DEG, HIDDEN), jnp.float32),
            pltpu.VMEM((HIDDEN,), jnp.float32),
            pltpu.SemaphoreType.DMA,
        ],
    )(v, sel, times_flat, thr16)


def _tc_body(v0_ref, times_ref, thr_ref, w0_ref, b0_ref, W_ref, B_ref,
             Wv_ref, bv_ref, term_ref):
    v0 = v0_ref[0:1, :]
    times = times_ref[...]
    maskf = (times <= thr_ref[...]).astype(jnp.float32)
    lin = times * w0_ref[...] + b0_ref[...]
    per = jnp.sin(times * W_ref[...] + B_ref[...])
    z = jnp.concatenate([lin, per], axis=1)
    zm = z * maskf
    t_v_sum = jnp.sum(
        jnp.dot(zm, Wv_ref[...], preferred_element_type=jnp.float32),
        axis=0, keepdims=True)
    cnt_in = jnp.sum(maskf)
    cnt0 = jnp.float32(DEG) - cnt_in
    term_ref[...] = t_v_sum + cnt_in * bv_ref[...] - cnt0 * v0


def _tc_time_term(v, times, thr, w0, b0, W, B, Wv, bv):
    n_nodes = v.shape[0]
    return pl.pallas_call(
        _tc_body,
        grid=(1,),
        in_specs=[
            pl.BlockSpec((8, HIDDEN), lambda i: (0, 0)),
            pl.BlockSpec((DEG, 1), lambda i: (0, 0)),
            pl.BlockSpec((1, 1), lambda i: (0, 0)),
            pl.BlockSpec((1, 1), lambda i: (0, 0)),
            pl.BlockSpec((1, 1), lambda i: (0, 0)),
            pl.BlockSpec((1, 15), lambda i: (0, 0)),
            pl.BlockSpec((1, 15), lambda i: (0, 0)),
            pl.BlockSpec((16, HIDDEN), lambda i: (0, 0)),
            pl.BlockSpec((1, HIDDEN), lambda i: (0, 0)),
        ],
        out_specs=pl.BlockSpec((1, HIDDEN), lambda i: (0, 0)),
        out_shape=jax.ShapeDtypeStruct((1, HIDDEN), jnp.float32),
    )(v, times, thr, w0, b0, W, B, Wv, bv)


def kernel(nid, k, q, v, t, neighbors, times,
           t2v_w0, t2v_b0, t2v_W, t2v_B, Wq, bq, Wk, bk, Wv, bv):
    del nid, k, q, Wq, bq, Wk, bk
    row_sum = _sc_gather_sum(
        v, neighbors.reshape(DEG).astype(jnp.int32),
        times.reshape(DEG).astype(jnp.float32),
        jnp.broadcast_to(t.reshape(1), (LANES,)))
    term = _tc_time_term(
        v, times.reshape(DEG, 1), t.reshape(1, 1), t2v_w0.reshape(1, 1),
        t2v_b0.reshape(1, 1), t2v_W.reshape(1, 15), t2v_B.reshape(1, 15),
        Wv, bv.reshape(1, HIDDEN))
    return row_sum.reshape(1, HIDDEN) + term


# speedup vs baseline: 1.8775x; 1.1093x over previous
import jax
import jax.numpy as jnp
from jax import lax
from jax.experimental import pallas as pl
from jax.experimental.pallas import tpu as pltpu
from jax.experimental.pallas import tpu_sc as plsc

DEG = 64
HIDDEN = 128
LANES = 16
N_CHUNKS = HIDDEN // LANES
NW = 4
ROWS_W = DEG // NW


def _sc_body(v_hbm, sel_hbm, times_hbm, thr_hbm, out_hbm,
             idx_v, times_v, thr_v, rows_v, out_v, sem):
    c = lax.axis_index("c")
    s = lax.axis_index("s")
    w = c * 2 + s

    @pl.when(s < NW // 2)
    def _():
        pltpu.sync_copy(sel_hbm.at[pl.ds(w * ROWS_W, ROWS_W)], idx_v)
        pltpu.sync_copy(times_hbm.at[pl.ds(w * ROWS_W, ROWS_W)], times_v)
        pltpu.sync_copy(thr_hbm, thr_v)
        idx_v[...] = jnp.where(times_v[...] <= thr_v[...], idx_v[...],
                               jnp.zeros((LANES,), jnp.int32))
        pltpu.async_copy(v_hbm.at[idx_v], rows_v, sem).wait()
        for j in range(N_CHUNKS):
            sl = pl.ds(j * LANES, LANES)
            a = rows_v[0, sl] + rows_v[2, sl]
            b = rows_v[1, sl] + rows_v[3, sl]
            for i in range(4, ROWS_W, 2):
                a = a + rows_v[i, sl]
                b = b + rows_v[i + 1, sl]
            out_v[sl] = a + b
        pltpu.sync_copy(out_v, out_hbm.at[w])


def _sc_gather_sum(v, sel, times_flat, thr16):
    mesh = plsc.VectorSubcoreMesh(core_axis_name="c", subcore_axis_name="s")
    return pl.kernel(
        _sc_body,
        out_type=jax.ShapeDtypeStruct((NW, HIDDEN), jnp.float32),
        mesh=mesh,
        scratch_types=[
            pltpu.VMEM((ROWS_W,), jnp.int32),
            pltpu.VMEM((ROWS_W,), jnp.float32),
            pltpu.VMEM((LANES,), jnp.float32),
            pltpu.VMEM((ROWS_W, HIDDEN), jnp.float32),
            pltpu.VMEM((HIDDEN,), jnp.float32),
            pltpu.SemaphoreType.DMA,
        ],
    )(v, sel, times_flat, thr16)


def _tc_body(v0_ref, times_ref, thr_ref, w0_ref, b0_ref, W_ref, B_ref,
             Wv_ref, bv_ref, term_ref):
    v0 = v0_ref[0:1, :]
    times = times_ref[...]
    maskf = (times <= thr_ref[...]).astype(jnp.float32)
    lin = times * w0_ref[...] + b0_ref[...]
    per = jnp.sin(times * W_ref[...] + B_ref[...])
    z = jnp.concatenate([lin, per], axis=1)
    zm = z * maskf
    t_v_sum = jnp.sum(
        jnp.dot(zm, Wv_ref[...], preferred_element_type=jnp.float32),
        axis=0, keepdims=True)
    cnt_in = jnp.sum(maskf)
    cnt0 = jnp.float32(DEG) - cnt_in
    term_ref[...] = t_v_sum + cnt_in * bv_ref[...] - cnt0 * v0


def _tc_time_term(v, times, thr, w0, b0, W, B, Wv, bv):
    return pl.pallas_call(
        _tc_body,
        grid=(1,),
        in_specs=[
            pl.BlockSpec((8, HIDDEN), lambda i: (0, 0)),
            pl.BlockSpec((DEG, 1), lambda i: (0, 0)),
            pl.BlockSpec((1, 1), lambda i: (0, 0)),
            pl.BlockSpec((1, 1), lambda i: (0, 0)),
            pl.BlockSpec((1, 1), lambda i: (0, 0)),
            pl.BlockSpec((1, 15), lambda i: (0, 0)),
            pl.BlockSpec((1, 15), lambda i: (0, 0)),
            pl.BlockSpec((16, HIDDEN), lambda i: (0, 0)),
            pl.BlockSpec((1, HIDDEN), lambda i: (0, 0)),
        ],
        out_specs=pl.BlockSpec((1, HIDDEN), lambda i: (0, 0)),
        out_shape=jax.ShapeDtypeStruct((1, HIDDEN), jnp.float32),
    )(v, times, thr, w0, b0, W, B, Wv, bv)


def kernel(nid, k, q, v, t, neighbors, times,
           t2v_w0, t2v_b0, t2v_W, t2v_B, Wq, bq, Wk, bk, Wv, bv):
    del nid, k, q, Wq, bq, Wk, bk
    parts = _sc_gather_sum(
        v, neighbors.reshape(DEG).astype(jnp.int32),
        times.reshape(DEG).astype(jnp.float32),
        jnp.broadcast_to(t.reshape(1), (LANES,)))
    term = _tc_time_term(
        v, times.reshape(DEG, 1), t.reshape(1, 1), t2v_w0.reshape(1, 1),
        t2v_b0.reshape(1, 1), t2v_W.reshape(1, 15), t2v_B.reshape(1, 15),
        Wv, bv.reshape(1, HIDDEN))
    row_sum = (parts[0] + parts[1]) + (parts[2] + parts[3])
    return row_sum.reshape(1, HIDDEN) + term
